# baseline (speedup 1.0000x reference)
import functools

import jax
import jax.numpy as jnp
import numpy as np
from jax import lax
from jax.experimental import pallas as pl
from jax.experimental.pallas import tpu as pltpu
from jax.experimental.pallas import tpu_sc as plsc

T = 2048
D = 768
F = 3072
E = 8
K = 2
BM = 128
NB = (T * K) // BM + E
NPAD = NB * BM
_C = 16
_R = T // _C


def _router_body(x_ref, wg_ref, q_ref, gate_ref, eblk_ref, act_ref,
                 bnd_ref, nxt_ref, par_ref, aux_ref):
    x = x_ref[...]
    logits = jnp.dot(x, wg_ref[...], preferred_element_type=jnp.float32)
    m = jnp.max(logits, axis=-1, keepdims=True)
    ex = jnp.exp(logits - m)
    probs = ex / jnp.sum(ex, axis=-1, keepdims=True)
    iota = jax.lax.broadcasted_iota(jnp.int32, probs.shape, 1)
    v0 = jnp.max(probs, axis=-1, keepdims=True)
    i0 = jnp.min(jnp.where(probs == v0, iota, E), axis=-1, keepdims=True)
    masked = jnp.where(iota == i0, -1.0, probs)
    v1 = jnp.max(masked, axis=-1, keepdims=True)
    i1 = jnp.min(jnp.where(masked == v1, iota, E), axis=-1, keepdims=True)
    s = v0 + v1
    gate_ref[...] = jnp.concatenate([v0 / s, v1 / s], axis=1)

    oh0 = (iota == i0).astype(jnp.float32)
    oh1 = (iota == i1).astype(jnp.float32)
    disp = oh0 + oh1

    ir = jax.lax.broadcasted_iota(jnp.int32, (_R, _R), 0)
    jr = jax.lax.broadcasted_iota(jnp.int32, (_R, _R), 1)
    tri_r = (ir > jr).astype(jnp.float32)
    locs = []
    csums = []
    for c in range(_C):
        dc = disp[c * _R:(c + 1) * _R, :]
        locs.append(jnp.dot(tri_r, dc, preferred_element_type=jnp.float32))
        csums.append(jnp.sum(dc, axis=0, keepdims=True))
    cs = jnp.concatenate(csums, axis=0)
    ic = jax.lax.broadcasted_iota(jnp.int32, (_C, _C), 0)
    jc = jax.lax.broadcasted_iota(jnp.int32, (_C, _C), 1)
    tri_c = (ic > jc).astype(jnp.float32)
    coffs = jnp.dot(tri_c, cs, preferred_element_type=jnp.float32)
    ranks = jnp.concatenate(
        [locs[c] + coffs[c:c + 1, :] for c in range(_C)], axis=0)

    counts = jnp.sum(cs, axis=0, keepdims=True)
    cnt_i = counts.astype(jnp.int32)
    nb_e = (cnt_i + (BM - 1)) // BM
    gs_pad = nb_e * BM
    offs, cnb = [jnp.zeros((1, 1), jnp.int32)], []
    run_o = gs_pad[:, 0:1]
    run_c = nb_e[:, 0:1]
    cnb.append(run_c)
    for e in range(1, E):
        offs.append(run_o)
        run_o = run_o + gs_pad[:, e:e + 1]
        run_c = run_c + nb_e[:, e:e + 1]
        cnb.append(run_c)
    offs_pad = jnp.concatenate(offs, axis=1).astype(jnp.float32)
    cum_nb = jnp.concatenate(cnb, axis=1)

    r0 = jnp.sum(ranks * oh0, axis=-1, keepdims=True)
    r1 = jnp.sum(ranks * oh1, axis=-1, keepdims=True)
    o0 = jnp.sum(offs_pad * oh0, axis=-1, keepdims=True)
    o1 = jnp.sum(offs_pad * oh1, axis=-1, keepdims=True)
    q0 = (r0 + o0).astype(jnp.int32)
    q1 = (r1 + o1).astype(jnp.int32)
    q_ref[...] = jnp.concatenate([q0, q1], axis=1)

    bik = jax.lax.broadcasted_iota(jnp.int32, (E, NB), 1)
    cnb_col = cum_nb.reshape(E, 1)
    e_blk = jnp.sum((bik >= cnb_col).astype(jnp.int32), axis=0, keepdims=True)
    total_nb = jnp.max(cum_nb)
    active = (jax.lax.broadcasted_iota(jnp.int32, (1, NB), 1)
              < total_nb).astype(jnp.int32)
    iota8 = jax.lax.broadcasted_iota(jnp.int32, (1, E), 1)
    last_e = jnp.max(jnp.where(cnt_i > 0, iota8, 0))
    e_blk = jnp.where(active > 0, jnp.minimum(e_blk, E - 1), last_e)
    eblk_ref[...] = e_blk
    act_ref[...] = active

    act_e = (nb_e > 0).astype(jnp.int32)
    ords, run_a = [], jnp.zeros((1, 1), jnp.int32)
    for e in range(E):
        ords.append(run_a)
        run_a = run_a + act_e[:, e:e + 1]
    ord_e = jnp.concatenate(ords, axis=1)
    nxts, c = [None] * E, jnp.full((1, 1), -1, jnp.int32)
    for e in range(E - 1, -1, -1):
        nxts[e] = c
        c = jnp.where(act_e[:, e:e + 1] > 0, jnp.full_like(c, e), c)
    nxt_of = jnp.concatenate(nxts, axis=1)
    ohb = (bik == e_blk).astype(jnp.int32)
    nxt_blk = jnp.sum(nxt_of.reshape(E, 1) * ohb, axis=0, keepdims=True)
    par_blk = jnp.sum((ord_e % 2).reshape(E, 1) * ohb, axis=0, keepdims=True)
    offs_blk = (cum_nb - nb_e).reshape(E, 1)
    ib = jax.lax.broadcasted_iota(jnp.int32, (E, NB), 1)
    bnd_blk = jnp.sum((ib == offs_blk).astype(jnp.int32)
                      * act_e.reshape(E, 1), axis=0, keepdims=True)
    nxt_ref[...] = nxt_blk
    par_ref[...] = par_blk
    bnd_ref[...] = bnd_blk

    psum = jnp.sum(probs, axis=0, keepdims=True)
    aux_ref[...] = (E / (T * T)) * jnp.sum(counts * psum, keepdims=True)


_router = pl.pallas_call(
    _router_body,
    out_shape=(
        jax.ShapeDtypeStruct((T, K), jnp.int32),
        jax.ShapeDtypeStruct((T, K), jnp.float32),
        jax.ShapeDtypeStruct((1, NB), jnp.int32),
        jax.ShapeDtypeStruct((1, NB), jnp.int32),
        jax.ShapeDtypeStruct((1, NB), jnp.int32),
        jax.ShapeDtypeStruct((1, NB), jnp.int32),
        jax.ShapeDtypeStruct((1, NB), jnp.int32),
        jax.ShapeDtypeStruct((1, 1), jnp.float32),
    ),
)


_NF = 4
_BF = F // _NF


def _gmm_body(e_ref, act_ref, x_hbm, w1_ref, b1_ref, w2_ref, b2_ref, y_ref,
              x_vm, y_acc, sem):
    j = pl.program_id(0)
    i = pl.program_id(1)

    @pl.when((j == 0) & (i == 0))
    def _():
        pltpu.make_async_copy(x_hbm, x_vm, sem).start()
        pltpu.make_async_copy(x_hbm, x_vm, sem).wait()

    @pl.when(act_ref[i] > 0)
    def _():
        sl = pl.ds(i * BM, BM)
        h = jnp.dot(x_vm[sl, :], w1_ref[0],
                    preferred_element_type=jnp.float32)
        h = jnp.maximum(h + b1_ref[0], 0.0)
        contrib = jnp.dot(h, w2_ref[0], preferred_element_type=jnp.float32)

        @pl.when(j == 0)
        def _():
            y_acc[sl, :] = contrib

        @pl.when(j > 0)
        def _():
            y_acc[sl, :] += contrib

        @pl.when(j == _NF - 1)
        def _():
            y_ref[...] = y_acc[sl, :] + b2_ref[0]


_gmm = pl.pallas_call(
    _gmm_body,
    grid_spec=pltpu.PrefetchScalarGridSpec(
        num_scalar_prefetch=2,
        grid=(_NF, NB),
        in_specs=[
            pl.BlockSpec(memory_space=pltpu.MemorySpace.HBM),
            pl.BlockSpec((1, D, _BF), lambda j, i, e, a: (e[i], 0, j)),
            pl.BlockSpec((1, 1, _BF), lambda j, i, e, a: (e[i], 0, j)),
            pl.BlockSpec((1, _BF, D), lambda j, i, e, a: (e[i], j, 0)),
            pl.BlockSpec((1, 1, D), lambda j, i, e, a: (e[i], 0, 0)),
        ],
        out_specs=pl.BlockSpec(
            (BM, D),
            lambda j, i, e, a: (jnp.where(j == _NF - 1, i, NB), 0)),
        scratch_shapes=[
            pltpu.VMEM((NPAD, D), jnp.float32),
            pltpu.VMEM((NPAD, D), jnp.float32),
            pltpu.SemaphoreType.DMA,
        ],
    ),
    out_shape=jax.ShapeDtypeStruct((NPAD + BM, D), jnp.float32),
)


def _ln_body(x_ref, m_ref, g_ref, gamma_ref, beta_ref, o_ref):
    g = g_ref[...]
    m = m_ref[...]
    z = (x_ref[...] + g[:, 0:1] * m[:, 0:D] + g[:, 1:2] * m[:, D:2 * D])
    mu = jnp.mean(z, axis=-1, keepdims=True)
    zc = z - mu
    var = jnp.mean(zc * zc, axis=-1, keepdims=True)
    o_ref[...] = zc * jax.lax.rsqrt(var + 1e-5) * gamma_ref[...] + beta_ref[...]


_LN_BM = 256
_ln = pl.pallas_call(
    _ln_body,
    grid=(T // _LN_BM,),
    in_specs=[
        pl.BlockSpec((_LN_BM, D), lambda i: (i, 0)),
        pl.BlockSpec((_LN_BM, K * D), lambda i: (i, 0)),
        pl.BlockSpec((_LN_BM, K), lambda i: (i, 0)),
        pl.BlockSpec((1, D), lambda i: (0, 0)),
        pl.BlockSpec((1, D), lambda i: (0, 0)),
    ],
    out_specs=pl.BlockSpec((_LN_BM, D), lambda i: (i, 0)),
    out_shape=jax.ShapeDtypeStruct((T, D), jnp.float32),
)


_SC_INFO = plsc.get_sparse_core_info()
_NC = _SC_INFO.num_cores
_NW = _NC * _SC_INFO.num_subcores
_A = T * K
_APW = _A // _NW
_TOK = np.arange(_A, dtype=np.int32) // K

_sc_mesh = plsc.VectorSubcoreMesh(core_axis_name="c", subcore_axis_name="s")


@functools.partial(
    pl.kernel, mesh=_sc_mesh,
    out_type=jax.ShapeDtypeStruct((NPAD, D), jnp.float32),
    scratch_types=[
        pltpu.VMEM((_APW,), jnp.int32),
        pltpu.VMEM((_APW,), jnp.int32),
        pltpu.VMEM((_APW, D), jnp.float32),
        pltpu.SemaphoreType.DMA,
    ],
)
def _sc_scatter_x(x_hbm, tok_hbm, q_hbm, xpad_hbm, tok_v, q_v, rows_v, sem):
    wid = lax.axis_index("s") * _NC + lax.axis_index("c")
    base = wid * _APW
    pltpu.sync_copy(tok_hbm.at[pl.ds(base, _APW)], tok_v)
    pltpu.sync_copy(q_hbm.at[pl.ds(base, _APW)], q_v)
    pltpu.async_copy(x_hbm.at[tok_v], rows_v, sem).wait()
    pltpu.async_copy(rows_v, xpad_hbm.at[q_v], sem).wait()


@functools.partial(
    pl.kernel, mesh=_sc_mesh,
    out_type=jax.ShapeDtypeStruct((_A, D), jnp.float32),
    scratch_types=[
        pltpu.VMEM((_APW,), jnp.int32),
        pltpu.VMEM((_APW, D), jnp.float32),
        pltpu.SemaphoreType.DMA,
    ],
)
def _sc_gather_y(ypad_hbm, q_hbm, out_hbm, q_v, rows_v, sem):
    wid = lax.axis_index("s") * _NC + lax.axis_index("c")
    base = wid * _APW
    pltpu.sync_copy(q_hbm.at[pl.ds(base, _APW)], q_v)
    pltpu.async_copy(ypad_hbm.at[q_v], rows_v, sem).wait()
    pltpu.sync_copy(rows_v, out_hbm.at[pl.ds(base, _APW)])


def kernel(x, mask, Wg, W1, b1, W2, b2, gamma, beta):
    del mask
    Bq, Sq, Dq = x.shape
    xf = x.reshape(T, D)

    q, gates, e_blk, active, bnd, nxt, par, aux = _router(xf, Wg)

    q_flat = q.reshape(-1)
    x_pad = _sc_scatter_x(xf, jnp.asarray(_TOK), q_flat)

    del bnd, nxt, par
    y_pad = _gmm(e_blk.reshape(NB), active.reshape(NB), x_pad, W1,
                 b1.reshape(E, 1, F), W2, b2.reshape(E, 1, D))[:NPAD]

    y_tok = _sc_gather_y(y_pad, q_flat).reshape(T, K * D)

    out = _ln(xf, y_tok, gates, gamma.reshape(1, D), beta.reshape(1, D))
    return out.reshape(Bq, Sq, Dq), aux[0, 0]


# speedup vs baseline: 1.5197x; 1.5197x over previous
import functools

import jax
import jax.numpy as jnp
import numpy as np
from jax import lax
from jax.experimental import pallas as pl
from jax.experimental.pallas import tpu as pltpu
from jax.experimental.pallas import tpu_sc as plsc

T = 2048
D = 768
F = 3072
E = 8
K = 2
BM = 256
NB = (T * K) // BM + E
NPAD = NB * BM
_C = 16
_R = T // _C


def _router_body(x_ref, wg_ref, q_ref, gate_ref, eblk_ref, act_ref,
                 bnd_ref, nxt_ref, par_ref, aux_ref):
    x = x_ref[...]
    logits = jnp.dot(x, wg_ref[...], preferred_element_type=jnp.float32)
    m = jnp.max(logits, axis=-1, keepdims=True)
    ex = jnp.exp(logits - m)
    probs = ex / jnp.sum(ex, axis=-1, keepdims=True)
    iota = jax.lax.broadcasted_iota(jnp.int32, probs.shape, 1)
    v0 = jnp.max(probs, axis=-1, keepdims=True)
    i0 = jnp.min(jnp.where(probs == v0, iota, E), axis=-1, keepdims=True)
    masked = jnp.where(iota == i0, -1.0, probs)
    v1 = jnp.max(masked, axis=-1, keepdims=True)
    i1 = jnp.min(jnp.where(masked == v1, iota, E), axis=-1, keepdims=True)
    s = v0 + v1
    gate_ref[...] = jnp.concatenate([v0 / s, v1 / s], axis=1)

    oh0 = (iota == i0).astype(jnp.float32)
    oh1 = (iota == i1).astype(jnp.float32)
    disp = oh0 + oh1

    ir = jax.lax.broadcasted_iota(jnp.int32, (_R, _R), 0)
    jr = jax.lax.broadcasted_iota(jnp.int32, (_R, _R), 1)
    tri_r = (ir > jr).astype(jnp.float32)
    locs = []
    csums = []
    for c in range(_C):
        dc = disp[c * _R:(c + 1) * _R, :]
        locs.append(jnp.dot(tri_r, dc, preferred_element_type=jnp.float32))
        csums.append(jnp.sum(dc, axis=0, keepdims=True))
    cs = jnp.concatenate(csums, axis=0)
    ic = jax.lax.broadcasted_iota(jnp.int32, (_C, _C), 0)
    jc = jax.lax.broadcasted_iota(jnp.int32, (_C, _C), 1)
    tri_c = (ic > jc).astype(jnp.float32)
    coffs = jnp.dot(tri_c, cs, preferred_element_type=jnp.float32)
    ranks = jnp.concatenate(
        [locs[c] + coffs[c:c + 1, :] for c in range(_C)], axis=0)

    counts = jnp.sum(cs, axis=0, keepdims=True)
    cnt_i = counts.astype(jnp.int32)
    nb_e = (cnt_i + (BM - 1)) // BM
    gs_pad = nb_e * BM
    offs, cnb = [jnp.zeros((1, 1), jnp.int32)], []
    run_o = gs_pad[:, 0:1]
    run_c = nb_e[:, 0:1]
    cnb.append(run_c)
    for e in range(1, E):
        offs.append(run_o)
        run_o = run_o + gs_pad[:, e:e + 1]
        run_c = run_c + nb_e[:, e:e + 1]
        cnb.append(run_c)
    offs_pad = jnp.concatenate(offs, axis=1).astype(jnp.float32)
    cum_nb = jnp.concatenate(cnb, axis=1)

    r0 = jnp.sum(ranks * oh0, axis=-1, keepdims=True)
    r1 = jnp.sum(ranks * oh1, axis=-1, keepdims=True)
    o0 = jnp.sum(offs_pad * oh0, axis=-1, keepdims=True)
    o1 = jnp.sum(offs_pad * oh1, axis=-1, keepdims=True)
    q0 = (r0 + o0).astype(jnp.int32)
    q1 = (r1 + o1).astype(jnp.int32)
    q_ref[...] = jnp.concatenate([q0, q1], axis=1)

    bik = jax.lax.broadcasted_iota(jnp.int32, (E, NB), 1)
    cnb_col = cum_nb.reshape(E, 1)
    e_blk = jnp.sum((bik >= cnb_col).astype(jnp.int32), axis=0, keepdims=True)
    total_nb = jnp.max(cum_nb)
    active = (jax.lax.broadcasted_iota(jnp.int32, (1, NB), 1)
              < total_nb).astype(jnp.int32)
    iota8 = jax.lax.broadcasted_iota(jnp.int32, (1, E), 1)
    last_e = jnp.max(jnp.where(cnt_i > 0, iota8, 0))
    e_blk = jnp.where(active > 0, jnp.minimum(e_blk, E - 1), last_e)
    eblk_ref[...] = e_blk
    act_ref[...] = active

    act_e = (nb_e > 0).astype(jnp.int32)
    ords, run_a = [], jnp.zeros((1, 1), jnp.int32)
    for e in range(E):
        ords.append(run_a)
        run_a = run_a + act_e[:, e:e + 1]
    ord_e = jnp.concatenate(ords, axis=1)
    nxts, c = [None] * E, jnp.full((1, 1), -1, jnp.int32)
    for e in range(E - 1, -1, -1):
        nxts[e] = c
        c = jnp.where(act_e[:, e:e + 1] > 0, jnp.full_like(c, e), c)
    nxt_of = jnp.concatenate(nxts, axis=1)
    ohb = (bik == e_blk).astype(jnp.int32)
    nxt_blk = jnp.sum(nxt_of.reshape(E, 1) * ohb, axis=0, keepdims=True)
    par_blk = jnp.sum((ord_e % 2).reshape(E, 1) * ohb, axis=0, keepdims=True)
    offs_blk = (cum_nb - nb_e).reshape(E, 1)
    ib = jax.lax.broadcasted_iota(jnp.int32, (E, NB), 1)
    bnd_blk = jnp.sum((ib == offs_blk).astype(jnp.int32)
                      * act_e.reshape(E, 1), axis=0, keepdims=True)
    nxt_ref[...] = nxt_blk
    par_ref[...] = par_blk
    bnd_ref[...] = bnd_blk

    psum = jnp.sum(probs, axis=0, keepdims=True)
    aux_ref[...] = (E / (T * T)) * jnp.sum(counts * psum, keepdims=True)


_router = pl.pallas_call(
    _router_body,
    out_shape=(
        jax.ShapeDtypeStruct((T, K), jnp.int32),
        jax.ShapeDtypeStruct((T, K), jnp.float32),
        jax.ShapeDtypeStruct((1, NB), jnp.int32),
        jax.ShapeDtypeStruct((1, NB), jnp.int32),
        jax.ShapeDtypeStruct((1, NB), jnp.int32),
        jax.ShapeDtypeStruct((1, NB), jnp.int32),
        jax.ShapeDtypeStruct((1, NB), jnp.int32),
        jax.ShapeDtypeStruct((1, 1), jnp.float32),
    ),
)


def _gmm_body(e_ref, act_ref, x_ref, w1_ref, b1_ref, w2_ref, b2_ref, y_ref):
    i = pl.program_id(0)

    @pl.when(act_ref[i] > 0)
    def _():
        h = jnp.dot(x_ref[...], w1_ref[0], preferred_element_type=jnp.float32)
        h = jnp.maximum(h + b1_ref[0], 0.0)
        y = jnp.dot(h, w2_ref[0], preferred_element_type=jnp.float32)
        y_ref[...] = y + b2_ref[0]


_gmm = pl.pallas_call(
    _gmm_body,
    grid_spec=pltpu.PrefetchScalarGridSpec(
        num_scalar_prefetch=2,
        grid=(NB,),
        in_specs=[
            pl.BlockSpec((BM, D), lambda i, e, a: (i, 0)),
            pl.BlockSpec((1, D, F), lambda i, e, a: (e[i], 0, 0)),
            pl.BlockSpec((1, 1, F), lambda i, e, a: (e[i], 0, 0)),
            pl.BlockSpec((1, F, D), lambda i, e, a: (e[i], 0, 0)),
            pl.BlockSpec((1, 1, D), lambda i, e, a: (e[i], 0, 0)),
        ],
        out_specs=pl.BlockSpec((BM, D), lambda i, e, a: (i, 0)),
    ),
    out_shape=jax.ShapeDtypeStruct((NPAD, D), jnp.float32),
)


def _ln_body(x_ref, m_ref, g_ref, gamma_ref, beta_ref, o_ref):
    g = g_ref[...]
    m = m_ref[...]
    z = (x_ref[...] + g[:, 0:1] * m[:, 0:D] + g[:, 1:2] * m[:, D:2 * D])
    mu = jnp.mean(z, axis=-1, keepdims=True)
    zc = z - mu
    var = jnp.mean(zc * zc, axis=-1, keepdims=True)
    o_ref[...] = zc * jax.lax.rsqrt(var + 1e-5) * gamma_ref[...] + beta_ref[...]


_LN_BM = 256
_ln = pl.pallas_call(
    _ln_body,
    grid=(T // _LN_BM,),
    in_specs=[
        pl.BlockSpec((_LN_BM, D), lambda i: (i, 0)),
        pl.BlockSpec((_LN_BM, K * D), lambda i: (i, 0)),
        pl.BlockSpec((_LN_BM, K), lambda i: (i, 0)),
        pl.BlockSpec((1, D), lambda i: (0, 0)),
        pl.BlockSpec((1, D), lambda i: (0, 0)),
    ],
    out_specs=pl.BlockSpec((_LN_BM, D), lambda i: (i, 0)),
    out_shape=jax.ShapeDtypeStruct((T, D), jnp.float32),
)


_SC_INFO = plsc.get_sparse_core_info()
_NC = _SC_INFO.num_cores
_NW = _NC * _SC_INFO.num_subcores
_A = T * K
_APW = _A // _NW
_TOK = np.arange(_A, dtype=np.int32) // K

_sc_mesh = plsc.VectorSubcoreMesh(core_axis_name="c", subcore_axis_name="s")


@functools.partial(
    pl.kernel, mesh=_sc_mesh,
    out_type=jax.ShapeDtypeStruct((NPAD, D), jnp.float32),
    scratch_types=[
        pltpu.VMEM((_APW,), jnp.int32),
        pltpu.VMEM((_APW,), jnp.int32),
        pltpu.VMEM((_APW, D), jnp.float32),
        pltpu.SemaphoreType.DMA,
    ],
)
def _sc_scatter_x(x_hbm, tok_hbm, q_hbm, xpad_hbm, tok_v, q_v, rows_v, sem):
    wid = lax.axis_index("s") * _NC + lax.axis_index("c")
    base = wid * _APW
    pltpu.sync_copy(tok_hbm.at[pl.ds(base, _APW)], tok_v)
    pltpu.sync_copy(q_hbm.at[pl.ds(base, _APW)], q_v)
    pltpu.async_copy(x_hbm.at[tok_v], rows_v, sem).wait()
    pltpu.async_copy(rows_v, xpad_hbm.at[q_v], sem).wait()


@functools.partial(
    pl.kernel, mesh=_sc_mesh,
    out_type=jax.ShapeDtypeStruct((_A, D), jnp.float32),
    scratch_types=[
        pltpu.VMEM((_APW,), jnp.int32),
        pltpu.VMEM((_APW, D), jnp.float32),
        pltpu.SemaphoreType.DMA,
    ],
)
def _sc_gather_y(ypad_hbm, q_hbm, out_hbm, q_v, rows_v, sem):
    wid = lax.axis_index("s") * _NC + lax.axis_index("c")
    base = wid * _APW
    pltpu.sync_copy(q_hbm.at[pl.ds(base, _APW)], q_v)
    pltpu.async_copy(ypad_hbm.at[q_v], rows_v, sem).wait()
    pltpu.sync_copy(rows_v, out_hbm.at[pl.ds(base, _APW)])


def kernel(x, mask, Wg, W1, b1, W2, b2, gamma, beta):
    del mask
    Bq, Sq, Dq = x.shape
    xf = x.reshape(T, D)

    q, gates, e_blk, active, bnd, nxt, par, aux = _router(xf, Wg)

    q_flat = q.reshape(-1)
    x_pad = _sc_scatter_x(xf, jnp.asarray(_TOK), q_flat)

    del bnd, nxt, par
    y_pad = _gmm(e_blk.reshape(NB), active.reshape(NB), x_pad, W1,
                 b1.reshape(E, 1, F), W2, b2.reshape(E, 1, D))

    y_tok = _sc_gather_y(y_pad, q_flat).reshape(T, K * D)

    out = _ln(xf, y_tok, gates, gamma.reshape(1, D), beta.reshape(1, D))
    return out.reshape(Bq, Sq, Dq), aux[0, 0]


# speedup vs baseline: 1.5208x; 1.0007x over previous
import functools

import jax
import jax.numpy as jnp
import numpy as np
from jax import lax
from jax.experimental import pallas as pl
from jax.experimental.pallas import tpu as pltpu
from jax.experimental.pallas import tpu_sc as plsc

T = 2048
D = 768
F = 3072
E = 8
K = 2
BM = 384
NB = (T * K) // BM + E
NPAD = NB * BM
_C = 16
_R = T // _C


def _router_body(x_ref, wg_ref, q_ref, gate_ref, eblk_ref, act_ref,
                 bnd_ref, nxt_ref, par_ref, aux_ref):
    x = x_ref[...]
    logits = jnp.dot(x, wg_ref[...], preferred_element_type=jnp.float32)
    m = jnp.max(logits, axis=-1, keepdims=True)
    ex = jnp.exp(logits - m)
    probs = ex / jnp.sum(ex, axis=-1, keepdims=True)
    iota = jax.lax.broadcasted_iota(jnp.int32, probs.shape, 1)
    v0 = jnp.max(probs, axis=-1, keepdims=True)
    i0 = jnp.min(jnp.where(probs == v0, iota, E), axis=-1, keepdims=True)
    masked = jnp.where(iota == i0, -1.0, probs)
    v1 = jnp.max(masked, axis=-1, keepdims=True)
    i1 = jnp.min(jnp.where(masked == v1, iota, E), axis=-1, keepdims=True)
    s = v0 + v1
    gate_ref[...] = jnp.concatenate([v0 / s, v1 / s], axis=1)

    oh0 = (iota == i0).astype(jnp.float32)
    oh1 = (iota == i1).astype(jnp.float32)
    disp = oh0 + oh1

    ir = jax.lax.broadcasted_iota(jnp.int32, (_R, _R), 0)
    jr = jax.lax.broadcasted_iota(jnp.int32, (_R, _R), 1)
    tri_r = (ir > jr).astype(jnp.float32)
    locs = []
    csums = []
    for c in range(_C):
        dc = disp[c * _R:(c + 1) * _R, :]
        locs.append(jnp.dot(tri_r, dc, preferred_element_type=jnp.float32))
        csums.append(jnp.sum(dc, axis=0, keepdims=True))
    cs = jnp.concatenate(csums, axis=0)
    ic = jax.lax.broadcasted_iota(jnp.int32, (_C, _C), 0)
    jc = jax.lax.broadcasted_iota(jnp.int32, (_C, _C), 1)
    tri_c = (ic > jc).astype(jnp.float32)
    coffs = jnp.dot(tri_c, cs, preferred_element_type=jnp.float32)
    ranks = jnp.concatenate(
        [locs[c] + coffs[c:c + 1, :] for c in range(_C)], axis=0)

    counts = jnp.sum(cs, axis=0, keepdims=True)
    cnt_i = counts.astype(jnp.int32)
    nb_e = (cnt_i + (BM - 1)) // BM
    gs_pad = nb_e * BM
    offs, cnb = [jnp.zeros((1, 1), jnp.int32)], []
    run_o = gs_pad[:, 0:1]
    run_c = nb_e[:, 0:1]
    cnb.append(run_c)
    for e in range(1, E):
        offs.append(run_o)
        run_o = run_o + gs_pad[:, e:e + 1]
        run_c = run_c + nb_e[:, e:e + 1]
        cnb.append(run_c)
    offs_pad = jnp.concatenate(offs, axis=1).astype(jnp.float32)
    cum_nb = jnp.concatenate(cnb, axis=1)

    r0 = jnp.sum(ranks * oh0, axis=-1, keepdims=True)
    r1 = jnp.sum(ranks * oh1, axis=-1, keepdims=True)
    o0 = jnp.sum(offs_pad * oh0, axis=-1, keepdims=True)
    o1 = jnp.sum(offs_pad * oh1, axis=-1, keepdims=True)
    q0 = (r0 + o0).astype(jnp.int32)
    q1 = (r1 + o1).astype(jnp.int32)
    q_ref[...] = jnp.concatenate([q0, q1], axis=1)

    bik = jax.lax.broadcasted_iota(jnp.int32, (E, NB), 1)
    cnb_col = cum_nb.reshape(E, 1)
    e_blk = jnp.sum((bik >= cnb_col).astype(jnp.int32), axis=0, keepdims=True)
    total_nb = jnp.max(cum_nb)
    active = (jax.lax.broadcasted_iota(jnp.int32, (1, NB), 1)
              < total_nb).astype(jnp.int32)
    iota8 = jax.lax.broadcasted_iota(jnp.int32, (1, E), 1)
    last_e = jnp.max(jnp.where(cnt_i > 0, iota8, 0))
    e_blk = jnp.where(active > 0, jnp.minimum(e_blk, E - 1), last_e)
    eblk_ref[...] = e_blk
    act_ref[...] = active

    act_e = (nb_e > 0).astype(jnp.int32)
    ords, run_a = [], jnp.zeros((1, 1), jnp.int32)
    for e in range(E):
        ords.append(run_a)
        run_a = run_a + act_e[:, e:e + 1]
    ord_e = jnp.concatenate(ords, axis=1)
    nxts, c = [None] * E, jnp.full((1, 1), -1, jnp.int32)
    for e in range(E - 1, -1, -1):
        nxts[e] = c
        c = jnp.where(act_e[:, e:e + 1] > 0, jnp.full_like(c, e), c)
    nxt_of = jnp.concatenate(nxts, axis=1)
    nxt_of = jnp.where(nxt_of < 0, iota8, nxt_of)
    ohb = (bik == e_blk).astype(jnp.int32)
    nxt_blk = jnp.sum(nxt_of.reshape(E, 1) * ohb, axis=0, keepdims=True)
    par_blk = jnp.sum((ord_e % 2).reshape(E, 1) * ohb, axis=0, keepdims=True)
    offs_blk = (cum_nb - nb_e).reshape(E, 1)
    ib = jax.lax.broadcasted_iota(jnp.int32, (E, NB), 1)
    bnd_blk = jnp.sum((ib == offs_blk).astype(jnp.int32)
                      * act_e.reshape(E, 1), axis=0, keepdims=True)
    nxt_ref[...] = nxt_blk
    par_ref[...] = par_blk
    bnd_ref[...] = bnd_blk

    psum = jnp.sum(probs, axis=0, keepdims=True)
    aux_ref[...] = (E / (T * T)) * jnp.sum(counts * psum, keepdims=True)


_router = pl.pallas_call(
    _router_body,
    out_shape=(
        jax.ShapeDtypeStruct((T, K), jnp.int32),
        jax.ShapeDtypeStruct((T, K), jnp.float32),
        jax.ShapeDtypeStruct((1, NB), jnp.int32),
        jax.ShapeDtypeStruct((1, NB), jnp.int32),
        jax.ShapeDtypeStruct((1, NB), jnp.int32),
        jax.ShapeDtypeStruct((1, NB), jnp.int32),
        jax.ShapeDtypeStruct((1, NB), jnp.int32),
        jax.ShapeDtypeStruct((1, 1), jnp.float32),
    ),
)


def _gmm_body(e_ref, act_ref, x_ref, w1_ref, b1_ref, w2_ref, b2_ref, y_ref):
    i = pl.program_id(0)

    @pl.when(act_ref[i] > 0)
    def _():
        h = jnp.dot(x_ref[...], w1_ref[0], preferred_element_type=jnp.float32)
        h = jnp.maximum(h + b1_ref[0], 0.0)
        y = jnp.dot(h, w2_ref[0], preferred_element_type=jnp.float32)
        y_ref[...] = y + b2_ref[0]


_gmm = pl.pallas_call(
    _gmm_body,
    grid_spec=pltpu.PrefetchScalarGridSpec(
        num_scalar_prefetch=2,
        grid=(NB,),
        in_specs=[
            pl.BlockSpec((BM, D), lambda i, e, a: (i, 0)),
            pl.BlockSpec((1, D, F), lambda i, e, a: (e[i], 0, 0)),
            pl.BlockSpec((1, 1, F), lambda i, e, a: (e[i], 0, 0)),
            pl.BlockSpec((1, F, D), lambda i, e, a: (e[i], 0, 0)),
            pl.BlockSpec((1, 1, D), lambda i, e, a: (e[i], 0, 0)),
        ],
        out_specs=pl.BlockSpec((BM, D), lambda i, e, a: (i, 0)),
    ),
    out_shape=jax.ShapeDtypeStruct((NPAD, D), jnp.float32),
)


def _ln_body(x_ref, m_ref, g_ref, gamma_ref, beta_ref, o_ref):
    g = g_ref[...]
    m = m_ref[...]
    z = (x_ref[...] + g[:, 0:1] * m[:, 0:D] + g[:, 1:2] * m[:, D:2 * D])
    mu = jnp.mean(z, axis=-1, keepdims=True)
    zc = z - mu
    var = jnp.mean(zc * zc, axis=-1, keepdims=True)
    o_ref[...] = zc * jax.lax.rsqrt(var + 1e-5) * gamma_ref[...] + beta_ref[...]


_LN_BM = 256
_ln = pl.pallas_call(
    _ln_body,
    grid=(T // _LN_BM,),
    in_specs=[
        pl.BlockSpec((_LN_BM, D), lambda i: (i, 0)),
        pl.BlockSpec((_LN_BM, K * D), lambda i: (i, 0)),
        pl.BlockSpec((_LN_BM, K), lambda i: (i, 0)),
        pl.BlockSpec((1, D), lambda i: (0, 0)),
        pl.BlockSpec((1, D), lambda i: (0, 0)),
    ],
    out_specs=pl.BlockSpec((_LN_BM, D), lambda i: (i, 0)),
    out_shape=jax.ShapeDtypeStruct((T, D), jnp.float32),
)


_SC_INFO = plsc.get_sparse_core_info()
_NC = _SC_INFO.num_cores
_NW = _NC * _SC_INFO.num_subcores
_A = T * K
_APW = _A // _NW
_TOK = np.arange(_A, dtype=np.int32) // K

_sc_mesh = plsc.VectorSubcoreMesh(core_axis_name="c", subcore_axis_name="s")


@functools.partial(
    pl.kernel, mesh=_sc_mesh,
    out_type=jax.ShapeDtypeStruct((NPAD, D), jnp.float32),
    scratch_types=[
        pltpu.VMEM((_APW,), jnp.int32),
        pltpu.VMEM((_APW,), jnp.int32),
        pltpu.VMEM((_APW, D), jnp.float32),
        pltpu.SemaphoreType.DMA,
    ],
)
def _sc_scatter_x(x_hbm, tok_hbm, q_hbm, xpad_hbm, tok_v, q_v, rows_v, sem):
    wid = lax.axis_index("s") * _NC + lax.axis_index("c")
    base = wid * _APW
    pltpu.sync_copy(tok_hbm.at[pl.ds(base, _APW)], tok_v)
    pltpu.sync_copy(q_hbm.at[pl.ds(base, _APW)], q_v)
    pltpu.async_copy(x_hbm.at[tok_v], rows_v, sem).wait()
    pltpu.async_copy(rows_v, xpad_hbm.at[q_v], sem).wait()


@functools.partial(
    pl.kernel, mesh=_sc_mesh,
    out_type=jax.ShapeDtypeStruct((_A, D), jnp.float32),
    scratch_types=[
        pltpu.VMEM((_APW,), jnp.int32),
        pltpu.VMEM((_APW, D), jnp.float32),
        pltpu.SemaphoreType.DMA,
    ],
)
def _sc_gather_y(ypad_hbm, q_hbm, out_hbm, q_v, rows_v, sem):
    wid = lax.axis_index("s") * _NC + lax.axis_index("c")
    base = wid * _APW
    pltpu.sync_copy(q_hbm.at[pl.ds(base, _APW)], q_v)
    pltpu.async_copy(ypad_hbm.at[q_v], rows_v, sem).wait()
    pltpu.sync_copy(rows_v, out_hbm.at[pl.ds(base, _APW)])


def kernel(x, mask, Wg, W1, b1, W2, b2, gamma, beta):
    del mask
    Bq, Sq, Dq = x.shape
    xf = x.reshape(T, D)

    q, gates, e_blk, active, bnd, nxt, par, aux = _router(xf, Wg)

    q_flat = q.reshape(-1)
    x_pad = _sc_scatter_x(xf, jnp.asarray(_TOK), q_flat)

    del bnd, nxt, par
    y_pad = _gmm(e_blk.reshape(NB), active.reshape(NB), x_pad, W1,
                 b1.reshape(E, 1, F), W2, b2.reshape(E, 1, D))

    y_tok = _sc_gather_y(y_pad, q_flat).reshape(T, K * D)

    out = _ln(xf, y_tok, gates, gamma.reshape(1, D), beta.reshape(1, D))
    return out.reshape(Bq, Sq, Dq), aux[0, 0]


# speedup vs baseline: 1.5224x; 1.0010x over previous
import functools

import jax
import jax.numpy as jnp
import numpy as np
from jax import lax
from jax.experimental import pallas as pl
from jax.experimental.pallas import tpu as pltpu
from jax.experimental.pallas import tpu_sc as plsc

T = 2048
D = 768
F = 3072
E = 8
K = 2
BM = 384
NB = (T * K) // BM + E
NPAD = NB * BM
_C = 16
_R = T // _C


def _router_body(x_ref, wg_ref, q_ref, gate_ref, eblk_ref, act_ref, aux_ref):
    x = x_ref[...]
    logits = jnp.dot(x, wg_ref[...], preferred_element_type=jnp.float32)
    m = jnp.max(logits, axis=-1, keepdims=True)
    ex = jnp.exp(logits - m)
    probs = ex / jnp.sum(ex, axis=-1, keepdims=True)
    iota = jax.lax.broadcasted_iota(jnp.int32, probs.shape, 1)
    v0 = jnp.max(probs, axis=-1, keepdims=True)
    i0 = jnp.min(jnp.where(probs == v0, iota, E), axis=-1, keepdims=True)
    masked = jnp.where(iota == i0, -1.0, probs)
    v1 = jnp.max(masked, axis=-1, keepdims=True)
    i1 = jnp.min(jnp.where(masked == v1, iota, E), axis=-1, keepdims=True)
    s = v0 + v1
    gate_ref[...] = jnp.concatenate([v0 / s, v1 / s], axis=1)

    oh0 = (iota == i0).astype(jnp.float32)
    oh1 = (iota == i1).astype(jnp.float32)
    disp = oh0 + oh1

    ir = jax.lax.broadcasted_iota(jnp.int32, (_R, _R), 0)
    jr = jax.lax.broadcasted_iota(jnp.int32, (_R, _R), 1)
    tri_r = (ir > jr).astype(jnp.float32)
    locs = []
    csums = []
    for c in range(_C):
        dc = disp[c * _R:(c + 1) * _R, :]
        locs.append(jnp.dot(tri_r, dc, preferred_element_type=jnp.float32))
        csums.append(jnp.sum(dc, axis=0, keepdims=True))
    cs = jnp.concatenate(csums, axis=0)
    ic = jax.lax.broadcasted_iota(jnp.int32, (_C, _C), 0)
    jc = jax.lax.broadcasted_iota(jnp.int32, (_C, _C), 1)
    tri_c = (ic > jc).astype(jnp.float32)
    coffs = jnp.dot(tri_c, cs, preferred_element_type=jnp.float32)
    ranks = jnp.concatenate(
        [locs[c] + coffs[c:c + 1, :] for c in range(_C)], axis=0)

    counts = jnp.sum(cs, axis=0, keepdims=True)
    cnt_i = counts.astype(jnp.int32)
    nb_e = (cnt_i + (BM - 1)) // BM
    gs_pad = nb_e * BM
    offs, cnb = [jnp.zeros((1, 1), jnp.int32)], []
    run_o = gs_pad[:, 0:1]
    run_c = nb_e[:, 0:1]
    cnb.append(run_c)
    for e in range(1, E):
        offs.append(run_o)
        run_o = run_o + gs_pad[:, e:e + 1]
        run_c = run_c + nb_e[:, e:e + 1]
        cnb.append(run_c)
    offs_pad = jnp.concatenate(offs, axis=1).astype(jnp.float32)
    cum_nb = jnp.concatenate(cnb, axis=1)

    r0 = jnp.sum(ranks * oh0, axis=-1, keepdims=True)
    r1 = jnp.sum(ranks * oh1, axis=-1, keepdims=True)
    o0 = jnp.sum(offs_pad * oh0, axis=-1, keepdims=True)
    o1 = jnp.sum(offs_pad * oh1, axis=-1, keepdims=True)
    q0 = (r0 + o0).astype(jnp.int32)
    q1 = (r1 + o1).astype(jnp.int32)
    q_ref[...] = jnp.concatenate([q0, q1], axis=1)

    bik = jax.lax.broadcasted_iota(jnp.int32, (E, NB), 1)
    cnb_col = cum_nb.reshape(E, 1)
    e_blk = jnp.sum((bik >= cnb_col).astype(jnp.int32), axis=0, keepdims=True)
    total_nb = jnp.max(cum_nb)
    active = (jax.lax.broadcasted_iota(jnp.int32, (1, NB), 1)
              < total_nb).astype(jnp.int32)
    iota8 = jax.lax.broadcasted_iota(jnp.int32, (1, E), 1)
    last_e = jnp.max(jnp.where(cnt_i > 0, iota8, 0))
    e_blk = jnp.where(active > 0, jnp.minimum(e_blk, E - 1), last_e)
    eblk_ref[...] = e_blk
    act_ref[...] = active

    psum = jnp.sum(probs, axis=0, keepdims=True)
    aux_ref[...] = (E / (T * T)) * jnp.sum(counts * psum, keepdims=True)


_router = pl.pallas_call(
    _router_body,
    out_shape=(
        jax.ShapeDtypeStruct((T, K), jnp.int32),
        jax.ShapeDtypeStruct((T, K), jnp.float32),
        jax.ShapeDtypeStruct((1, NB), jnp.int32),
        jax.ShapeDtypeStruct((1, NB), jnp.int32),
        jax.ShapeDtypeStruct((1, 1), jnp.float32),
    ),
)


def _gmm_body(e_ref, act_ref, x_ref, w1_ref, b1_ref, w2_ref, b2_ref, y_ref):
    i = pl.program_id(0)

    @pl.when(act_ref[i] > 0)
    def _():
        h = jnp.dot(x_ref[...], w1_ref[0], preferred_element_type=jnp.float32)
        h = jnp.maximum(h + b1_ref[0], 0.0)
        y = jnp.dot(h, w2_ref[0], preferred_element_type=jnp.float32)
        y_ref[...] = y + b2_ref[0]


_gmm = pl.pallas_call(
    _gmm_body,
    grid_spec=pltpu.PrefetchScalarGridSpec(
        num_scalar_prefetch=2,
        grid=(NB,),
        in_specs=[
            pl.BlockSpec((BM, D), lambda i, e, a: (i, 0)),
            pl.BlockSpec((1, D, F), lambda i, e, a: (e[i], 0, 0)),
            pl.BlockSpec((1, 1, F), lambda i, e, a: (e[i], 0, 0)),
            pl.BlockSpec((1, F, D), lambda i, e, a: (e[i], 0, 0)),
            pl.BlockSpec((1, 1, D), lambda i, e, a: (e[i], 0, 0)),
        ],
        out_specs=pl.BlockSpec((BM, D), lambda i, e, a: (i, 0)),
    ),
    out_shape=jax.ShapeDtypeStruct((NPAD, D), jnp.float32),
)


def _ln_body(x_ref, m_ref, g_ref, gamma_ref, beta_ref, o_ref):
    g = g_ref[...]
    m = m_ref[...]
    z = (x_ref[...] + g[:, 0:1] * m[:, 0:D] + g[:, 1:2] * m[:, D:2 * D])
    mu = jnp.mean(z, axis=-1, keepdims=True)
    zc = z - mu
    var = jnp.mean(zc * zc, axis=-1, keepdims=True)
    o_ref[...] = zc * jax.lax.rsqrt(var + 1e-5) * gamma_ref[...] + beta_ref[...]


_LN_BM = 256
_ln = pl.pallas_call(
    _ln_body,
    grid=(T // _LN_BM,),
    in_specs=[
        pl.BlockSpec((_LN_BM, D), lambda i: (i, 0)),
        pl.BlockSpec((_LN_BM, K * D), lambda i: (i, 0)),
        pl.BlockSpec((_LN_BM, K), lambda i: (i, 0)),
        pl.BlockSpec((1, D), lambda i: (0, 0)),
        pl.BlockSpec((1, D), lambda i: (0, 0)),
    ],
    out_specs=pl.BlockSpec((_LN_BM, D), lambda i: (i, 0)),
    out_shape=jax.ShapeDtypeStruct((T, D), jnp.float32),
)


_SC_INFO = plsc.get_sparse_core_info()
_NC = _SC_INFO.num_cores
_NW = _NC * _SC_INFO.num_subcores
_A = T * K
_APW = _A // _NW
_TOK = np.arange(_A, dtype=np.int32) // K

_sc_mesh = plsc.VectorSubcoreMesh(core_axis_name="c", subcore_axis_name="s")


@functools.partial(
    pl.kernel, mesh=_sc_mesh,
    out_type=jax.ShapeDtypeStruct((NPAD, D), jnp.float32),
    scratch_types=[
        pltpu.VMEM((_APW,), jnp.int32),
        pltpu.VMEM((_APW,), jnp.int32),
        pltpu.VMEM((_APW, D), jnp.float32),
        pltpu.SemaphoreType.DMA,
    ],
)
def _sc_scatter_x(x_hbm, tok_hbm, q_hbm, xpad_hbm, tok_v, q_v, rows_v, sem):
    wid = lax.axis_index("s") * _NC + lax.axis_index("c")
    base = wid * _APW
    pltpu.sync_copy(tok_hbm.at[pl.ds(base, _APW)], tok_v)
    pltpu.sync_copy(q_hbm.at[pl.ds(base, _APW)], q_v)
    pltpu.async_copy(x_hbm.at[tok_v], rows_v, sem).wait()
    pltpu.async_copy(rows_v, xpad_hbm.at[q_v], sem).wait()


@functools.partial(
    pl.kernel, mesh=_sc_mesh,
    out_type=jax.ShapeDtypeStruct((_A, D), jnp.float32),
    scratch_types=[
        pltpu.VMEM((_APW,), jnp.int32),
        pltpu.VMEM((_APW, D), jnp.float32),
        pltpu.SemaphoreType.DMA,
    ],
)
def _sc_gather_y(ypad_hbm, q_hbm, out_hbm, q_v, rows_v, sem):
    wid = lax.axis_index("s") * _NC + lax.axis_index("c")
    base = wid * _APW
    pltpu.sync_copy(q_hbm.at[pl.ds(base, _APW)], q_v)
    pltpu.async_copy(ypad_hbm.at[q_v], rows_v, sem).wait()
    pltpu.sync_copy(rows_v, out_hbm.at[pl.ds(base, _APW)])


def kernel(x, mask, Wg, W1, b1, W2, b2, gamma, beta):
    del mask
    Bq, Sq, Dq = x.shape
    xf = x.reshape(T, D)

    q, gates, e_blk, active, aux = _router(xf, Wg)

    q_flat = q.reshape(-1)
    x_pad = _sc_scatter_x(xf, jnp.asarray(_TOK), q_flat)

    y_pad = _gmm(e_blk.reshape(NB), active.reshape(NB), x_pad, W1,
                 b1.reshape(E, 1, F), W2, b2.reshape(E, 1, D))

    y_tok = _sc_gather_y(y_pad, q_flat).reshape(T, K * D)

    out = _ln(xf, y_tok, gates, gamma.reshape(1, D), beta.reshape(1, D))
    return out.reshape(Bq, Sq, Dq), aux[0, 0]
